# baseline (speedup 1.0000x reference)
import jax
import jax.numpy as jnp
from jax.experimental import pallas as pl
from jax.experimental.pallas import tpu as pltpu

_BB = 32
_BM = 25088


def _mm_kernel(params_ref, x_ref, tx_ref, mem_ref, out_t_ref, out_ref):
    inv_t = 1.0 / params_ref[0]
    xx = jnp.concatenate([x_ref[...], tx_ref[...]], axis=0) * inv_t
    y = jax.lax.dot_general(
        xx, mem_ref[...], (((1,), (1,)), ((), ())),
        preferred_element_type=jnp.float32)
    out_ref[...] = y[:_BB]
    out_t_ref[...] = y[_BB:]


@jax.jit
def kernel(image_features, transformed_image_features, indices, memory, params):
    del indices
    B, D = image_features.shape
    M = memory.shape[0]
    grid = (pl.cdiv(M, _BM), B // _BB)
    out_shape = jax.ShapeDtypeStruct((B, M), jnp.float32)
    out_t, out = pl.pallas_call(
        _mm_kernel,
        grid=grid,
        in_specs=[
            pl.BlockSpec(memory_space=pltpu.SMEM),
            pl.BlockSpec((_BB, D), lambda j, i: (i, 0)),
            pl.BlockSpec((_BB, D), lambda j, i: (i, 0)),
            pl.BlockSpec((_BM, D), lambda j, i: (j, 0)),
        ],
        out_specs=[
            pl.BlockSpec((_BB, _BM), lambda j, i: (i, j)),
            pl.BlockSpec((_BB, _BM), lambda j, i: (i, j)),
        ],
        out_shape=[out_shape, out_shape],
        compiler_params=pltpu.CompilerParams(
            dimension_semantics=("arbitrary", "arbitrary"),
        ),
    )(params, image_features, transformed_image_features, memory)
    return (out_t, out)


# speedup vs baseline: 1.1439x; 1.1439x over previous
import jax
import jax.numpy as jnp
from jax.experimental import pallas as pl
from jax.experimental.pallas import tpu as pltpu

_BB = 16
_NSPLIT = 4
_ROWS = _BB // _NSPLIT


def _mm_kernel(params_ref, x_ref, tx_ref, memt_ref, out_t_ref, out_ref,
               scratch, sems):
    i = pl.program_id(0)
    nsteps = pl.num_programs(0)
    slot = jax.lax.rem(i, 2)
    inv_t = 1.0 / params_ref[0]

    def copies(s_idx, step):
        cps = []
        for o_idx, oref in ((0, out_ref), (1, out_t_ref)):
            for s in range(_NSPLIT):
                src = scratch.at[s_idx, pl.ds(o_idx * _BB + s * _ROWS, _ROWS)]
                dst = oref.at[pl.ds(step * _BB + s * _ROWS, _ROWS)]
                cps.append(pltpu.make_async_copy(src, dst,
                                                 sems.at[s_idx, o_idx, s]))
        return cps

    @pl.when(i >= 2)
    def _wait_prev():
        for cp in copies(slot, i - 2):
            cp.wait()

    xx = jnp.concatenate([x_ref[...], tx_ref[...]], axis=0) * inv_t
    y = jax.lax.dot_general(
        xx, memt_ref[...], (((1,), (0,)), ((), ())),
        preferred_element_type=jnp.float32)
    scratch[slot] = y

    for cp in copies(slot, i):
        cp.start()

    @pl.when(i == nsteps - 1)
    def _drain():
        for cp in copies(1 - slot, i - 1):
            cp.wait()
        for cp in copies(slot, i):
            cp.wait()


@jax.jit
def kernel(image_features, transformed_image_features, indices, memory, params):
    del indices
    B, D = image_features.shape
    M = memory.shape[0]
    mem_t = memory.T
    grid = (B // _BB,)
    out_shape = jax.ShapeDtypeStruct((B, M), jnp.float32)
    out_t, out = pl.pallas_call(
        _mm_kernel,
        grid=grid,
        in_specs=[
            pl.BlockSpec(memory_space=pltpu.SMEM),
            pl.BlockSpec((_BB, D), lambda i: (i, 0)),
            pl.BlockSpec((_BB, D), lambda i: (i, 0)),
            pl.BlockSpec((D, M), lambda i: (0, 0)),
        ],
        out_specs=[
            pl.BlockSpec(memory_space=pltpu.MemorySpace.HBM),
            pl.BlockSpec(memory_space=pltpu.MemorySpace.HBM),
        ],
        out_shape=[out_shape, out_shape],
        scratch_shapes=[
            pltpu.VMEM((2, 2 * _BB, M), jnp.float32),
            pltpu.SemaphoreType.DMA((2, 2, _NSPLIT)),
        ],
        compiler_params=pltpu.CompilerParams(
            dimension_semantics=("arbitrary",),
        ),
    )(params, image_features, transformed_image_features, mem_t)
    return (out_t, out)
